# baseline (speedup 1.0000x reference)
import jax
import jax.numpy as jnp
from jax import lax
from jax.experimental import pallas as pl
from jax.experimental.pallas import tpu as pltpu
from jax.experimental.pallas import tpu_sc as plsc

NUM_EXPERTS = 64
TOP_K = 8
HIDDEN = 1024
FF = 768
T = 128

_NEG = -3.0e38
_LANES = 16
_WORKERS = T // _LANES
_EPB = 2


def _logits_body(x_ref, rw_ref, lgt_ref):
    lgt = lax.dot_general(rw_ref[...], x_ref[...], (((1,), (1,)), ((), ())),
                          preferred_element_type=jnp.float32)
    for w in range(_WORKERS):
        lgt_ref[w] = lgt[:, w * _LANES:(w + 1) * _LANES]


def _sc_combine_body(lgt_hbm, combt_hbm, vt_ref, ob_ref):
    wid = lax.axis_index("s") * 2 + lax.axis_index("c")

    @pl.when(wid < _WORKERS)
    def _():
        pltpu.sync_copy(lgt_hbm.at[wid], vt_ref)
        neg = jnp.full((_LANES,), _NEG, jnp.float32)
        vals = []
        idxs = []
        for k in range(TOP_K):
            def scan_body(e, carry):
                best, bidx = carry
                v = vt_ref[e, :]
                gt = v > best
                return (jnp.where(gt, v, best), jnp.where(gt, e, bidx))
            best, bidx = lax.fori_loop(
                0, NUM_EXPERTS, scan_body,
                (neg, jnp.zeros((_LANES,), jnp.int32)), unroll=8)
            vals.append(best)
            idxs.append(bidx)
            if k + 1 < TOP_K:
                def mask_body(e, _):
                    vt_ref[e, :] = jnp.where(bidx == e, neg, vt_ref[e, :])
                    return 0
                lax.fori_loop(0, NUM_EXPERTS, mask_body, 0, unroll=8)
        m = vals[0]
        exs = [jnp.exp(v - m) for v in vals]
        s = exs[0]
        for ex in exs[1:]:
            s = s + ex
        inv = 1.0 / s
        ws = [ex * inv for ex in exs]

        def out_body(e, _):
            w = jnp.zeros((_LANES,), jnp.float32)
            for k in range(TOP_K):
                w = w + jnp.where(idxs[k] == e, ws[k], 0.0)
            ob_ref[e, :] = w
            return 0
        lax.fori_loop(0, NUM_EXPERTS, out_body, 0, unroll=8)
        pltpu.sync_copy(ob_ref, combt_hbm.at[wid])


def _expert_body(x_ref, combt_ref, lgt_ref, g_ref, u_ref, d_ref,
                 out_ref, lg_ref, comb_ref):
    e = pl.program_id(0)

    @pl.when(e == 0)
    def _untranspose():
        eye = (lax.broadcasted_iota(jnp.int32, (_LANES, _LANES), 0)
               == lax.broadcasted_iota(jnp.int32, (_LANES, _LANES), 1)
               ).astype(jnp.float32)
        for w in range(_WORKERS):
            comb_ref[w * _LANES:(w + 1) * _LANES, :] = lax.dot_general(
                eye, combt_ref[w], (((1,), (1,)), ((), ())),
                preferred_element_type=jnp.float32)
            lg_ref[w * _LANES:(w + 1) * _LANES, :] = lax.dot_general(
                eye, lgt_ref[w], (((1,), (1,)), ((), ())),
                preferred_element_type=jnp.float32)

    x = x_ref[...]
    for i in range(_EPB):
        ei = e * _EPB + i
        g = lax.dot_general(x, g_ref[i], (((1,), (1,)), ((), ())),
                            preferred_element_type=jnp.float32)
        u = lax.dot_general(x, u_ref[i], (((1,), (1,)), ((), ())),
                            preferred_element_type=jnp.float32)
        h = g * (1.0 / (1.0 + jnp.exp(-g))) * u
        onehot = (lax.broadcasted_iota(jnp.int32, (NUM_EXPERTS, 1), 0) == ei
                  ).astype(jnp.float32)
        col = lax.dot_general(comb_ref[...], onehot, (((1,), (0,)), ((), ())),
                              preferred_element_type=jnp.float32)
        h = h * col
        contrib = lax.dot_general(h, d_ref[i], (((1,), (1,)), ((), ())),
                                  preferred_element_type=jnp.float32)

        if i == 0:
            @pl.when(e == 0)
            def _init():
                out_ref[...] = contrib

            @pl.when(e > 0)
            def _acc():
                out_ref[...] += contrib
        else:
            out_ref[...] += contrib


def kernel(hidden_states, router_weight, gate_proj, up_proj, down_proj):
    B, S, D = hidden_states.shape
    x = hidden_states.reshape(-1, D)

    logits_t = pl.pallas_call(
        _logits_body,
        out_shape=jax.ShapeDtypeStruct((_WORKERS, NUM_EXPERTS, _LANES),
                                       jnp.float32),
    )(x, router_weight)

    combine_t = pl.kernel(
        _sc_combine_body,
        out_type=jax.ShapeDtypeStruct((_WORKERS, NUM_EXPERTS, _LANES),
                                      jnp.float32),
        mesh=plsc.VectorSubcoreMesh(core_axis_name="c", subcore_axis_name="s"),
        scratch_types=[
            pltpu.VMEM((NUM_EXPERTS, _LANES), jnp.float32),
            pltpu.VMEM((NUM_EXPERTS, _LANES), jnp.float32),
        ],
    )(logits_t)

    out, logits = pl.pallas_call(
        _expert_body,
        grid=(NUM_EXPERTS // _EPB,),
        in_specs=[
            pl.BlockSpec((T, HIDDEN), lambda e: (0, 0)),
            pl.BlockSpec((_WORKERS, NUM_EXPERTS, _LANES),
                         lambda e: (0, 0, 0)),
            pl.BlockSpec((_WORKERS, NUM_EXPERTS, _LANES),
                         lambda e: (0, 0, 0)),
            pl.BlockSpec((_EPB, FF, HIDDEN), lambda e: (e, 0, 0)),
            pl.BlockSpec((_EPB, FF, HIDDEN), lambda e: (e, 0, 0)),
            pl.BlockSpec((_EPB, HIDDEN, FF), lambda e: (e, 0, 0)),
        ],
        out_specs=[
            pl.BlockSpec((T, HIDDEN), lambda e: (0, 0)),
            pl.BlockSpec((T, NUM_EXPERTS), lambda e: (0, 0)),
        ],
        out_shape=[
            jax.ShapeDtypeStruct((T, HIDDEN), jnp.float32),
            jax.ShapeDtypeStruct((T, NUM_EXPERTS), jnp.float32),
        ],
        scratch_shapes=[pltpu.VMEM((T, NUM_EXPERTS), jnp.float32)],
    )(x, combine_t, logits_t, gate_proj, up_proj, down_proj)

    return out.reshape(B, S, D), logits


# speedup vs baseline: 1.0075x; 1.0075x over previous
import jax
import jax.numpy as jnp
from jax import lax
from jax.experimental import pallas as pl
from jax.experimental.pallas import tpu as pltpu
from jax.experimental.pallas import tpu_sc as plsc

NUM_EXPERTS = 64
TOP_K = 8
HIDDEN = 1024
FF = 768
T = 128

_NEG = -3.0e38
_LANES = 16
_WORKERS = T // _LANES
_EPB = 2


def _logits_body(x_ref, rw_ref, lgt_ref):
    lgt = lax.dot_general(rw_ref[...], x_ref[...], (((1,), (1,)), ((), ())),
                          preferred_element_type=jnp.float32)
    for w in range(_WORKERS):
        lgt_ref[w] = lgt[:, w * _LANES:(w + 1) * _LANES]


def _sc_combine_body(lgt_hbm, combt_hbm, vt_ref, ob_ref):
    wid = lax.axis_index("s") * 2 + lax.axis_index("c")

    @pl.when(wid < _WORKERS)
    def _():
        pltpu.sync_copy(lgt_hbm.at[wid], vt_ref)
        neg = jnp.full((_LANES,), _NEG, jnp.float32)
        vals = []
        idxs = []
        for k in range(TOP_K):
            def scan_body(e, carry):
                best, bidx = carry
                v = vt_ref[e, :]
                gt = v > best
                return (jnp.where(gt, v, best), jnp.where(gt, e, bidx))
            best, bidx = lax.fori_loop(
                0, NUM_EXPERTS, scan_body,
                (neg, jnp.zeros((_LANES,), jnp.int32)), unroll=4)
            vals.append(best)
            idxs.append(bidx)
            if k + 1 < TOP_K:
                def mask_body(e, _):
                    vt_ref[e, :] = jnp.where(bidx == e, neg, vt_ref[e, :])
                    return 0
                lax.fori_loop(0, NUM_EXPERTS, mask_body, 0, unroll=4)
        m = vals[0]
        exs = [jnp.exp(v - m) for v in vals]
        s = exs[0]
        for ex in exs[1:]:
            s = s + ex
        inv = 1.0 / s
        ws = [ex * inv for ex in exs]

        def out_body(e, _):
            w = jnp.zeros((_LANES,), jnp.float32)
            for k in range(TOP_K):
                w = w + jnp.where(idxs[k] == e, ws[k], 0.0)
            ob_ref[e, :] = w
            return 0
        lax.fori_loop(0, NUM_EXPERTS, out_body, 0, unroll=4)
        pltpu.sync_copy(ob_ref, combt_hbm.at[wid])


def _expert_body(x_ref, combt_ref, lgt_ref, g_ref, u_ref, d_ref,
                 out_ref, lg_ref, comb_ref):
    e = pl.program_id(0)

    @pl.when(e == 0)
    def _untranspose():
        eye = (lax.broadcasted_iota(jnp.int32, (_LANES, _LANES), 0)
               == lax.broadcasted_iota(jnp.int32, (_LANES, _LANES), 1)
               ).astype(jnp.float32)
        for w in range(_WORKERS):
            comb_ref[w * _LANES:(w + 1) * _LANES, :] = lax.dot_general(
                eye, combt_ref[w], (((1,), (1,)), ((), ())),
                preferred_element_type=jnp.float32)
            lg_ref[w * _LANES:(w + 1) * _LANES, :] = lax.dot_general(
                eye, lgt_ref[w], (((1,), (1,)), ((), ())),
                preferred_element_type=jnp.float32)

    x = x_ref[...]
    for i in range(_EPB):
        ei = e * _EPB + i
        g = lax.dot_general(x, g_ref[i], (((1,), (1,)), ((), ())),
                            preferred_element_type=jnp.float32)
        u = lax.dot_general(x, u_ref[i], (((1,), (1,)), ((), ())),
                            preferred_element_type=jnp.float32)
        h = g * (1.0 / (1.0 + jnp.exp(-g))) * u
        onehot = (lax.broadcasted_iota(jnp.int32, (NUM_EXPERTS, 1), 0) == ei
                  ).astype(jnp.float32)
        col = lax.dot_general(comb_ref[...], onehot, (((1,), (0,)), ((), ())),
                              preferred_element_type=jnp.float32)
        h = h * col
        contrib = lax.dot_general(h, d_ref[i], (((1,), (1,)), ((), ())),
                                  preferred_element_type=jnp.float32)

        if i == 0:
            @pl.when(e == 0)
            def _init():
                out_ref[...] = contrib

            @pl.when(e > 0)
            def _acc():
                out_ref[...] += contrib
        else:
            out_ref[...] += contrib


def kernel(hidden_states, router_weight, gate_proj, up_proj, down_proj):
    B, S, D = hidden_states.shape
    x = hidden_states.reshape(-1, D)

    logits_t = pl.pallas_call(
        _logits_body,
        out_shape=jax.ShapeDtypeStruct((_WORKERS, NUM_EXPERTS, _LANES),
                                       jnp.float32),
    )(x, router_weight)

    combine_t = pl.kernel(
        _sc_combine_body,
        out_type=jax.ShapeDtypeStruct((_WORKERS, NUM_EXPERTS, _LANES),
                                      jnp.float32),
        mesh=plsc.VectorSubcoreMesh(core_axis_name="c", subcore_axis_name="s"),
        scratch_types=[
            pltpu.VMEM((NUM_EXPERTS, _LANES), jnp.float32),
            pltpu.VMEM((NUM_EXPERTS, _LANES), jnp.float32),
        ],
    )(logits_t)

    out, logits = pl.pallas_call(
        _expert_body,
        grid=(NUM_EXPERTS // _EPB,),
        in_specs=[
            pl.BlockSpec((T, HIDDEN), lambda e: (0, 0)),
            pl.BlockSpec((_WORKERS, NUM_EXPERTS, _LANES),
                         lambda e: (0, 0, 0)),
            pl.BlockSpec((_WORKERS, NUM_EXPERTS, _LANES),
                         lambda e: (0, 0, 0)),
            pl.BlockSpec((_EPB, FF, HIDDEN), lambda e: (e, 0, 0)),
            pl.BlockSpec((_EPB, FF, HIDDEN), lambda e: (e, 0, 0)),
            pl.BlockSpec((_EPB, HIDDEN, FF), lambda e: (e, 0, 0)),
        ],
        out_specs=[
            pl.BlockSpec((T, HIDDEN), lambda e: (0, 0)),
            pl.BlockSpec((T, NUM_EXPERTS), lambda e: (0, 0)),
        ],
        out_shape=[
            jax.ShapeDtypeStruct((T, HIDDEN), jnp.float32),
            jax.ShapeDtypeStruct((T, NUM_EXPERTS), jnp.float32),
        ],
        scratch_shapes=[pltpu.VMEM((T, NUM_EXPERTS), jnp.float32)],
    )(x, combine_t, logits_t, gate_proj, up_proj, down_proj)

    return out.reshape(B, S, D), logits
